# baseline (speedup 1.0000x reference)
import functools
import jax
import jax.numpy as jnp
from jax import lax
from jax.experimental import pallas as pl
from jax.experimental.pallas import tpu as pltpu
from jax.experimental.pallas import tpu_sc as plsc

D = 64
TEXT_VOCAB = 100000
CHAR_VOCAB = 1000
B, L = 4096, 200
NW = 32
BT = B // NW
LP = L // 2
BC = 256


def _mm_body(x_ref, w_ref, o_ref):
    o_ref[:] = jnp.dot(x_ref[:], w_ref[:], preferred_element_type=jnp.float32) * 8.0


def _mm_bias_body(x_ref, w_ref, b_ref, o_ref):
    o_ref[:] = (
        jnp.dot(x_ref[:], w_ref[:], preferred_element_type=jnp.float32) * 8.0
        + b_ref[:]
    )


def _fold_text_table(text_table, Wt):
    blk = 4000
    return pl.pallas_call(
        _mm_body,
        grid=(TEXT_VOCAB // blk,),
        in_specs=[
            pl.BlockSpec((blk, D), lambda i: (i, 0)),
            pl.BlockSpec((D, D), lambda i: (0, 0)),
        ],
        out_specs=pl.BlockSpec((blk, D), lambda i: (i, 0)),
        out_shape=jax.ShapeDtypeStruct((TEXT_VOCAB, D), jnp.float32),
    )(text_table, Wt)


def _fold_char_table(char_table, Wc, b2):
    return pl.pallas_call(
        _mm_bias_body,
        out_shape=jax.ShapeDtypeStruct((CHAR_VOCAB, D), jnp.float32),
    )(char_table, Wc, b2)


def _sc_body(t_hbm, c_hbm, idx_hbm, chars_hbm, y_hbm,
             idx_v, cbuf_v, cidx_v, rowsA, rowsB,
             semA, semB, semWA, semWB):
    wid = lax.axis_index("s") * 2 + lax.axis_index("c")

    pltpu.sync_copy(idx_hbm.at[pl.ds(wid * L, L)], idx_v)
    pltpu.sync_copy(chars_hbm.at[pl.ds(wid * BT, BT)], cidx_v)
    pltpu.async_copy(c_hbm.at[cidx_v], cbuf_v, semA).wait()

    b0 = wid * BT

    def fire_gather(l, rows_v, sem):
        pltpu.async_copy(t_hbm.at[idx_v.at[l]], rows_v, sem)

    def wait_gather(rows_v, sem):
        pltpu.make_async_copy(t_hbm.at[idx_v.at[0]], rows_v, sem).wait()

    def add_char(rows_v):
        @plsc.parallel_loop(0, BT, unroll=4)
        def _(s):
            for j in range(4):
                sl = pl.ds(j * 16, 16)
                rows_v[s, sl] = rows_v[s, sl] + cbuf_v[s, sl]

    def fire_write(l, rows_v, sem):
        pltpu.async_copy(
            rows_v, y_hbm.at[l // 2, pl.ds(b0, BT), pl.ds((l % 2) * D, D)], sem)

    def wait_write(rows_v, sem):
        pltpu.make_async_copy(
            rows_v, y_hbm.at[0, pl.ds(b0, BT), pl.ds(0, D)], sem).wait()

    fire_gather(0, rowsA, semA)
    fire_gather(1, rowsB, semB)

    def body(i, carry):
        lA = 2 * i
        lB = 2 * i + 1
        wait_gather(rowsA, semA)

        @pl.when(i > 0)
        def _():
            wait_write(rowsA, semWA)

        add_char(rowsA)
        fire_write(lA, rowsA, semWA)
        fire_gather(jnp.minimum(lA + 2, L - 1), rowsA, semA)
        wait_gather(rowsB, semB)

        @pl.when(i > 0)
        def _():
            wait_write(rowsB, semWB)

        add_char(rowsB)
        fire_write(lB, rowsB, semWB)
        fire_gather(jnp.minimum(lB + 2, L - 1), rowsB, semB)
        return carry

    lax.fori_loop(0, L // 2, body, 0)

    wait_gather(rowsA, semA)
    wait_gather(rowsB, semB)
    wait_write(rowsA, semWA)
    wait_write(rowsB, semWB)


def _sc_gather_add(T, C, IDX, chars):
    mesh = plsc.VectorSubcoreMesh(core_axis_name="c", subcore_axis_name="s")
    f = functools.partial(
        pl.kernel,
        mesh=mesh,
        compiler_params=pltpu.CompilerParams(
            use_tc_tiling_on_sc=False,
            disable_bounds_checks=True,
        ),
        out_type=jax.ShapeDtypeStruct((LP, B, 2 * D), jnp.float32),
        scratch_types=[
            pltpu.VMEM((L, BT), jnp.int32),
            pltpu.VMEM((BT, D), jnp.float32),
            pltpu.VMEM((BT,), jnp.int32),
            pltpu.VMEM((BT, D), jnp.float32),
            pltpu.VMEM((BT, D), jnp.float32),
            pltpu.SemaphoreType.DMA,
            pltpu.SemaphoreType.DMA,
            pltpu.SemaphoreType.DMA,
            pltpu.SemaphoreType.DMA,
        ],
    )(_sc_body)
    return f(T, C, IDX, chars)


def _tr_body(y_ref, o_ref):
    x = y_ref[0]
    o_ref[0] = x[:, :D].T
    o_ref[1] = x[:, D:].T


def _tc_transpose(Y):
    return pl.pallas_call(
        _tr_body,
        grid=(LP, B // BC),
        in_specs=[pl.BlockSpec((1, BC, 2 * D), lambda i, j: (i, j, 0))],
        out_specs=pl.BlockSpec((2, D, BC), lambda i, j: (i, 0, j)),
        out_shape=jax.ShapeDtypeStruct((L, D, B), jnp.float32),
    )(Y)


def kernel(text_seqs, chars, text_table, char_table, W, b):
    Wt = W[:D]
    Wc = W[D:]
    T = _fold_text_table(text_table, Wt)
    Cb = _fold_char_table(char_table, Wc, b.reshape(1, D))
    IDX = (
        text_seqs.astype(jnp.int32)
        .reshape(NW, BT, L)
        .transpose(0, 2, 1)
        .reshape(NW * L, BT)
    )
    Y = _sc_gather_add(T, Cb, IDX, chars.astype(jnp.int32))
    out_t = _tc_transpose(Y)
    return out_t.transpose(2, 0, 1)


# speedup vs baseline: 1.8749x; 1.8749x over previous
import functools
import jax
import jax.numpy as jnp
from jax import lax
from jax.experimental import pallas as pl
from jax.experimental.pallas import tpu as pltpu
from jax.experimental.pallas import tpu_sc as plsc

D = 64
TEXT_VOCAB = 100000
CHAR_VOCAB = 1000
B, L = 4096, 200
NW = 32
BT = B // NW
LP = L // 2
BC = 1024


def _mm_body(x_ref, w_ref, o_ref):
    o_ref[:] = jnp.dot(x_ref[:], w_ref[:], preferred_element_type=jnp.float32) * 8.0


def _mm_bias_body(x_ref, w_ref, b_ref, o_ref):
    o_ref[:] = (
        jnp.dot(x_ref[:], w_ref[:], preferred_element_type=jnp.float32) * 8.0
        + b_ref[:]
    )


def _fold_text_table(text_table, Wt):
    blk = 4000
    return pl.pallas_call(
        _mm_body,
        grid=(TEXT_VOCAB // blk,),
        in_specs=[
            pl.BlockSpec((blk, D), lambda i: (i, 0)),
            pl.BlockSpec((D, D), lambda i: (0, 0)),
        ],
        out_specs=pl.BlockSpec((blk, D), lambda i: (i, 0)),
        out_shape=jax.ShapeDtypeStruct((TEXT_VOCAB, D), jnp.float32),
    )(text_table, Wt)


def _fold_char_table(char_table, Wc, b2):
    return pl.pallas_call(
        _mm_bias_body,
        out_shape=jax.ShapeDtypeStruct((CHAR_VOCAB, D), jnp.float32),
    )(char_table, Wc, b2)


def _sc_body(t_hbm, c_hbm, idx_hbm, chars_hbm, y_hbm,
             idx_v, cbuf_v, cidx_v, rowsA, rowsB,
             semA, semB, semWA, semWB):
    wid = lax.axis_index("s") * 2 + lax.axis_index("c")

    pltpu.sync_copy(idx_hbm.at[pl.ds(wid * L, L)], idx_v)
    pltpu.sync_copy(chars_hbm.at[pl.ds(wid * BT, BT)], cidx_v)
    pltpu.async_copy(c_hbm.at[cidx_v], cbuf_v, semA).wait()

    b0 = wid * BT

    def fire_gather(l, rows_v, sem):
        pltpu.async_copy(t_hbm.at[idx_v.at[l]], rows_v, sem)

    def wait_gather(rows_v, sem):
        pltpu.make_async_copy(t_hbm.at[idx_v.at[0]], rows_v, sem).wait()

    def add_char(rows_v):
        @plsc.parallel_loop(0, BT, unroll=4)
        def _(s):
            for j in range(4):
                sl = pl.ds(j * 16, 16)
                rows_v[s, sl] = rows_v[s, sl] + cbuf_v[s, sl]

    def fire_write(l, rows_v, sem):
        pltpu.async_copy(
            rows_v, y_hbm.at[l // 2, pl.ds(b0, BT), pl.ds((l % 2) * D, D)], sem)

    def wait_write(rows_v, sem):
        pltpu.make_async_copy(
            rows_v, y_hbm.at[0, pl.ds(b0, BT), pl.ds(0, D)], sem).wait()

    fire_gather(0, rowsA, semA)
    fire_gather(1, rowsB, semB)

    def body(i, carry):
        lA = 2 * i
        lB = 2 * i + 1
        wait_gather(rowsA, semA)

        @pl.when(i > 0)
        def _():
            wait_write(rowsA, semWA)

        add_char(rowsA)
        fire_write(lA, rowsA, semWA)
        fire_gather(jnp.minimum(lA + 2, L - 1), rowsA, semA)
        wait_gather(rowsB, semB)

        @pl.when(i > 0)
        def _():
            wait_write(rowsB, semWB)

        add_char(rowsB)
        fire_write(lB, rowsB, semWB)
        fire_gather(jnp.minimum(lB + 2, L - 1), rowsB, semB)
        return carry

    lax.fori_loop(0, L // 2, body, 0)

    wait_gather(rowsA, semA)
    wait_gather(rowsB, semB)
    wait_write(rowsA, semWA)
    wait_write(rowsB, semWB)


def _sc_gather_add(T, C, IDX, chars):
    mesh = plsc.VectorSubcoreMesh(core_axis_name="c", subcore_axis_name="s")
    f = functools.partial(
        pl.kernel,
        mesh=mesh,
        compiler_params=pltpu.CompilerParams(
            use_tc_tiling_on_sc=False,
            disable_bounds_checks=True,
        ),
        out_type=jax.ShapeDtypeStruct((LP, B, 2 * D), jnp.float32),
        scratch_types=[
            pltpu.VMEM((L, BT), jnp.int32),
            pltpu.VMEM((BT, D), jnp.float32),
            pltpu.VMEM((BT,), jnp.int32),
            pltpu.VMEM((BT, D), jnp.float32),
            pltpu.VMEM((BT, D), jnp.float32),
            pltpu.SemaphoreType.DMA,
            pltpu.SemaphoreType.DMA,
            pltpu.SemaphoreType.DMA,
            pltpu.SemaphoreType.DMA,
        ],
    )(_sc_body)
    return f(T, C, IDX, chars)


def _tr_body(y_ref, o_ref):
    x = y_ref[0]
    o_ref[0] = x[:, :D].T
    o_ref[1] = x[:, D:].T


def _tc_transpose(Y):
    return pl.pallas_call(
        _tr_body,
        grid=(LP, B // BC),
        in_specs=[pl.BlockSpec((1, BC, 2 * D), lambda i, j: (i, j, 0))],
        out_specs=pl.BlockSpec((2, D, BC), lambda i, j: (i, 0, j)),
        out_shape=jax.ShapeDtypeStruct((L, D, B), jnp.float32),
    )(Y)


def kernel(text_seqs, chars, text_table, char_table, W, b):
    Wt = W[:D]
    Wc = W[D:]
    T = _fold_text_table(text_table, Wt)
    Cb = _fold_char_table(char_table, Wc, b.reshape(1, D))
    IDX = (
        text_seqs.astype(jnp.int32)
        .reshape(NW, BT, L)
        .transpose(0, 2, 1)
        .reshape(NW * L, BT)
    )
    Y = _sc_gather_add(T, Cb, IDX, chars.astype(jnp.int32))
    out_t = _tc_transpose(Y)
    return out_t.transpose(2, 0, 1)


# speedup vs baseline: 2.4040x; 1.2822x over previous
import functools
import jax
import jax.numpy as jnp
from jax import lax
from jax.experimental import pallas as pl
from jax.experimental.pallas import tpu as pltpu
from jax.experimental.pallas import tpu_sc as plsc

D = 64
TEXT_VOCAB = 100000
CHAR_VOCAB = 1000
B, L = 4096, 200
NW = 32
BT = B // NW
LP = L // 2
BC = 4096


def _mm_body(x_ref, w_ref, o_ref):
    o_ref[:] = jnp.dot(x_ref[:], w_ref[:], preferred_element_type=jnp.float32) * 8.0


def _mm_bias_body(x_ref, w_ref, b_ref, o_ref):
    o_ref[:] = (
        jnp.dot(x_ref[:], w_ref[:], preferred_element_type=jnp.float32) * 8.0
        + b_ref[:]
    )


def _fold_text_table(text_table, Wt):
    blk = 4000
    return pl.pallas_call(
        _mm_body,
        grid=(TEXT_VOCAB // blk,),
        in_specs=[
            pl.BlockSpec((blk, D), lambda i: (i, 0)),
            pl.BlockSpec((D, D), lambda i: (0, 0)),
        ],
        out_specs=pl.BlockSpec((blk, D), lambda i: (i, 0)),
        out_shape=jax.ShapeDtypeStruct((TEXT_VOCAB, D), jnp.float32),
    )(text_table, Wt)


def _fold_char_table(char_table, Wc, b2):
    return pl.pallas_call(
        _mm_bias_body,
        out_shape=jax.ShapeDtypeStruct((CHAR_VOCAB, D), jnp.float32),
    )(char_table, Wc, b2)


def _sc_body(t_hbm, c_hbm, idx_hbm, chars_hbm, y_hbm,
             idx_v, cbuf_v, cidx_v, rowsA, rowsB,
             semA, semB, semWA, semWB):
    wid = lax.axis_index("s") * 2 + lax.axis_index("c")

    pltpu.sync_copy(idx_hbm.at[pl.ds(wid * L, L)], idx_v)
    pltpu.sync_copy(chars_hbm.at[pl.ds(wid * BT, BT)], cidx_v)
    pltpu.async_copy(c_hbm.at[cidx_v], cbuf_v, semA).wait()

    b0 = wid * BT

    def fire_gather(l, rows_v, sem):
        pltpu.async_copy(t_hbm.at[idx_v.at[l]], rows_v, sem)

    def wait_gather(rows_v, sem):
        pltpu.make_async_copy(t_hbm.at[idx_v.at[0]], rows_v, sem).wait()

    def add_char(rows_v):
        @plsc.parallel_loop(0, BT, unroll=4)
        def _(s):
            for j in range(4):
                sl = pl.ds(j * 16, 16)
                rows_v[s, sl] = rows_v[s, sl] + cbuf_v[s, sl]

    def fire_write(l, rows_v, sem):
        pltpu.async_copy(
            rows_v, y_hbm.at[l // 2, pl.ds(b0, BT), pl.ds((l % 2) * D, D)], sem)

    def wait_write(rows_v, sem):
        pltpu.make_async_copy(
            rows_v, y_hbm.at[0, pl.ds(b0, BT), pl.ds(0, D)], sem).wait()

    fire_gather(0, rowsA, semA)
    fire_gather(1, rowsB, semB)

    def body(i, carry):
        lA = 2 * i
        lB = 2 * i + 1
        wait_gather(rowsA, semA)

        @pl.when(i > 0)
        def _():
            wait_write(rowsA, semWA)

        add_char(rowsA)
        fire_write(lA, rowsA, semWA)
        fire_gather(jnp.minimum(lA + 2, L - 1), rowsA, semA)
        wait_gather(rowsB, semB)

        @pl.when(i > 0)
        def _():
            wait_write(rowsB, semWB)

        add_char(rowsB)
        fire_write(lB, rowsB, semWB)
        fire_gather(jnp.minimum(lB + 2, L - 1), rowsB, semB)
        return carry

    lax.fori_loop(0, L // 2, body, 0)

    wait_gather(rowsA, semA)
    wait_gather(rowsB, semB)
    wait_write(rowsA, semWA)
    wait_write(rowsB, semWB)


def _sc_gather_add(T, C, IDX, chars):
    mesh = plsc.VectorSubcoreMesh(core_axis_name="c", subcore_axis_name="s")
    f = functools.partial(
        pl.kernel,
        mesh=mesh,
        compiler_params=pltpu.CompilerParams(
            use_tc_tiling_on_sc=False,
            disable_bounds_checks=True,
        ),
        out_type=jax.ShapeDtypeStruct((LP, B, 2 * D), jnp.float32),
        scratch_types=[
            pltpu.VMEM((L, BT), jnp.int32),
            pltpu.VMEM((BT, D), jnp.float32),
            pltpu.VMEM((BT,), jnp.int32),
            pltpu.VMEM((BT, D), jnp.float32),
            pltpu.VMEM((BT, D), jnp.float32),
            pltpu.SemaphoreType.DMA,
            pltpu.SemaphoreType.DMA,
            pltpu.SemaphoreType.DMA,
            pltpu.SemaphoreType.DMA,
        ],
    )(_sc_body)
    return f(T, C, IDX, chars)


def _tr_body(y_ref, o_ref):
    x = y_ref[0]
    o_ref[0] = x[:, :D].T
    o_ref[1] = x[:, D:].T


def _tc_transpose(Y):
    return pl.pallas_call(
        _tr_body,
        grid=(LP, B // BC),
        in_specs=[pl.BlockSpec((1, BC, 2 * D), lambda i, j: (i, j, 0))],
        out_specs=pl.BlockSpec((2, D, BC), lambda i, j: (i, 0, j)),
        out_shape=jax.ShapeDtypeStruct((L, D, B), jnp.float32),
    )(Y)


def kernel(text_seqs, chars, text_table, char_table, W, b):
    Wt = W[:D]
    Wc = W[D:]
    T = _fold_text_table(text_table, Wt)
    Cb = _fold_char_table(char_table, Wc, b.reshape(1, D))
    IDX = (
        text_seqs.astype(jnp.int32)
        .reshape(NW, BT, L)
        .transpose(0, 2, 1)
        .reshape(NW * L, BT)
    )
    Y = _sc_gather_add(T, Cb, IDX, chars.astype(jnp.int32))
    out_t = _tc_transpose(Y)
    return out_t.transpose(2, 0, 1)


# speedup vs baseline: 2.5440x; 1.0583x over previous
import functools
import jax
import jax.numpy as jnp
from jax import lax
from jax.experimental import pallas as pl
from jax.experimental.pallas import tpu as pltpu
from jax.experimental.pallas import tpu_sc as plsc

D = 64
TEXT_VOCAB = 100000
CHAR_VOCAB = 1000
B, L = 4096, 200
NW = 32
BT = B // NW
LP = L // 2
BC = 4096


def _mm_body(x_ref, w_ref, o_ref):
    o_ref[:] = jnp.dot(x_ref[:], w_ref[:], preferred_element_type=jnp.float32) * 8.0


def _mm_bias_body(x_ref, w_ref, b_ref, o_ref):
    o_ref[:] = (
        jnp.dot(x_ref[:], w_ref[:], preferred_element_type=jnp.float32) * 8.0
        + b_ref[:]
    )


def _fold_text_table(text_table, Wt):
    blk = 4000
    return pl.pallas_call(
        _mm_body,
        grid=(TEXT_VOCAB // blk,),
        in_specs=[
            pl.BlockSpec((blk, D), lambda i: (i, 0)),
            pl.BlockSpec((D, D), lambda i: (0, 0)),
        ],
        out_specs=pl.BlockSpec((blk, D), lambda i: (i, 0)),
        out_shape=jax.ShapeDtypeStruct((TEXT_VOCAB, D), jnp.float32),
    )(text_table, Wt)


def _fold_char_table(char_table, Wc, b2):
    return pl.pallas_call(
        _mm_bias_body,
        out_shape=jax.ShapeDtypeStruct((CHAR_VOCAB, D), jnp.float32),
    )(char_table, Wc, b2)


def _sc_body(t_hbm, c_hbm, idx_hbm, chars_hbm, y_hbm,
             idx_v, cbuf_v, cidx_v, rowsA, rowsB,
             semA, semB, semWA, semWB):
    wid = lax.axis_index("s") * 2 + lax.axis_index("c")

    pltpu.sync_copy(idx_hbm.at[pl.ds(wid * L, L)], idx_v)
    pltpu.sync_copy(chars_hbm.at[pl.ds(wid * BT, BT)], cidx_v)
    pltpu.async_copy(c_hbm.at[cidx_v], cbuf_v, semA).wait()

    b0 = wid * BT

    def fire_gather(l, rows_v, sem):
        pltpu.async_copy(t_hbm.at[idx_v.at[l]], rows_v, sem)

    def wait_gather(rows_v, sem):
        pltpu.make_async_copy(t_hbm.at[idx_v.at[0]], rows_v, sem).wait()

    def add_char(rows_v):
        @plsc.parallel_loop(0, BT, unroll=4)
        def _(s):
            for j in range(4):
                sl = pl.ds(j * 16, 16)
                rows_v[s, sl] = rows_v[s, sl] + cbuf_v[s, sl]

    def fire_write(l, rows_v, sem):
        pltpu.async_copy(
            rows_v, y_hbm.at[l // 2, pl.ds(b0, BT), pl.ds((l % 2) * D, D)], sem)

    def wait_write(rows_v, sem):
        pltpu.make_async_copy(
            rows_v, y_hbm.at[0, pl.ds(b0, BT), pl.ds(0, D)], sem).wait()

    fire_gather(0, rowsA, semA)
    fire_gather(1, rowsB, semB)

    def body(i, carry):
        lA = 2 * i
        lB = 2 * i + 1
        wait_gather(rowsA, semA)

        @pl.when(i > 0)
        def _():
            wait_write(rowsA, semWA)

        add_char(rowsA)
        fire_write(lA, rowsA, semWA)
        fire_gather(jnp.minimum(lA + 2, L - 1), rowsA, semA)
        wait_gather(rowsB, semB)

        @pl.when(i > 0)
        def _():
            wait_write(rowsB, semWB)

        add_char(rowsB)
        fire_write(lB, rowsB, semWB)
        fire_gather(jnp.minimum(lB + 2, L - 1), rowsB, semB)
        return carry

    lax.fori_loop(0, L // 2, body, 0)

    wait_gather(rowsA, semA)
    wait_gather(rowsB, semB)
    wait_write(rowsA, semWA)
    wait_write(rowsB, semWB)


def _sc_gather_add(T, C, IDX, chars):
    mesh = plsc.VectorSubcoreMesh(core_axis_name="c", subcore_axis_name="s")
    f = functools.partial(
        pl.kernel,
        mesh=mesh,
        compiler_params=pltpu.CompilerParams(
            use_tc_tiling_on_sc=False,
            disable_bounds_checks=True,
        ),
        out_type=jax.ShapeDtypeStruct((LP, B, 2 * D), jnp.float32),
        scratch_types=[
            pltpu.VMEM((L, BT), jnp.int32),
            pltpu.VMEM((BT, D), jnp.float32),
            pltpu.VMEM((BT,), jnp.int32),
            pltpu.VMEM((BT, D), jnp.float32),
            pltpu.VMEM((BT, D), jnp.float32),
            pltpu.SemaphoreType.DMA,
            pltpu.SemaphoreType.DMA,
            pltpu.SemaphoreType.DMA,
            pltpu.SemaphoreType.DMA,
        ],
    )(_sc_body)
    return f(T, C, IDX, chars)


def _tr_body(y_ref, o_ref):
    for p in range(2):
        x = y_ref[p]
        o_ref[2 * p] = x[:, :D].T
        o_ref[2 * p + 1] = x[:, D:].T


def _tc_transpose(Y):
    return pl.pallas_call(
        _tr_body,
        grid=(LP // 2,),
        in_specs=[pl.BlockSpec((2, BC, 2 * D), lambda i: (i, 0, 0))],
        out_specs=pl.BlockSpec((4, D, BC), lambda i: (i, 0, 0)),
        out_shape=jax.ShapeDtypeStruct((L, D, B), jnp.float32),
    )(Y)


def kernel(text_seqs, chars, text_table, char_table, W, b):
    Wt = W[:D]
    Wc = W[D:]
    T = _fold_text_table(text_table, Wt)
    Cb = _fold_char_table(char_table, Wc, b.reshape(1, D))
    IDX = (
        text_seqs.astype(jnp.int32)
        .reshape(NW, BT, L)
        .transpose(0, 2, 1)
        .reshape(NW * L, BT)
    )
    Y = _sc_gather_add(T, Cb, IDX, chars.astype(jnp.int32))
    out_t = _tc_transpose(Y)
    return out_t.transpose(2, 0, 1)


# speedup vs baseline: 2.6000x; 1.0220x over previous
import functools
import jax
import jax.numpy as jnp
from jax import lax
from jax.experimental import pallas as pl
from jax.experimental.pallas import tpu as pltpu
from jax.experimental.pallas import tpu_sc as plsc

D = 64
TEXT_VOCAB = 100000
CHAR_VOCAB = 1000
B, L = 4096, 200
NW = 32
BT = B // NW
LP = L // 2
BC = 4096


def _mm_body(x_ref, w_ref, o_ref):
    o_ref[:] = jnp.dot(x_ref[:], w_ref[:], preferred_element_type=jnp.float32) * 8.0


def _mm_bias_body(x_ref, w_ref, b_ref, o_ref):
    o_ref[:] = (
        jnp.dot(x_ref[:], w_ref[:], preferred_element_type=jnp.float32) * 8.0
        + b_ref[:]
    )


def _fold_text_table(text_table, Wt):
    blk = 4000
    return pl.pallas_call(
        _mm_body,
        grid=(TEXT_VOCAB // blk,),
        in_specs=[
            pl.BlockSpec((blk, D), lambda i: (i, 0)),
            pl.BlockSpec((D, D), lambda i: (0, 0)),
        ],
        out_specs=pl.BlockSpec((blk, D), lambda i: (i, 0)),
        out_shape=jax.ShapeDtypeStruct((TEXT_VOCAB, D), jnp.float32),
    )(text_table, Wt)


def _fold_char_table(char_table, Wc, b2):
    return pl.pallas_call(
        _mm_bias_body,
        out_shape=jax.ShapeDtypeStruct((CHAR_VOCAB, D), jnp.float32),
    )(char_table, Wc, b2)


def _sc_body(t_hbm, c_hbm, idx_hbm, chars_hbm, y_hbm,
             idx_v, cbuf_v, cidx_v, rowsA, rowsB,
             semA, semB, semWA, semWB):
    wid = lax.axis_index("s") * 2 + lax.axis_index("c")

    pltpu.sync_copy(idx_hbm.at[pl.ds(wid * L, L)], idx_v)
    pltpu.sync_copy(chars_hbm.at[pl.ds(wid * BT, BT)], cidx_v)
    pltpu.async_copy(c_hbm.at[cidx_v], cbuf_v, semA).wait()

    b0 = wid * BT

    def fire_gather(l, rows_v, sem):
        pltpu.async_copy(t_hbm.at[idx_v.at[l]], rows_v, sem)

    def wait_gather(rows_v, sem):
        pltpu.make_async_copy(t_hbm.at[idx_v.at[0]], rows_v, sem).wait()

    def add_char(rows_v):
        @plsc.parallel_loop(0, BT, unroll=4)
        def _(s):
            for j in range(4):
                sl = pl.ds(j * 16, 16)
                rows_v[s, sl] = rows_v[s, sl] + cbuf_v[s, sl]

    def fire_write(l, rows_v, sem):
        pltpu.async_copy(
            rows_v, y_hbm.at[l // 2, pl.ds(b0, BT), pl.ds((l % 2) * D, D)], sem)

    def wait_write(rows_v, sem):
        pltpu.make_async_copy(
            rows_v, y_hbm.at[0, pl.ds(b0, BT), pl.ds(0, D)], sem).wait()

    fire_gather(0, rowsA, semA)
    fire_gather(1, rowsB, semB)

    def body(i, carry):
        lA = 2 * i
        lB = 2 * i + 1
        wait_gather(rowsA, semA)

        @pl.when(i > 0)
        def _():
            wait_write(rowsA, semWA)

        add_char(rowsA)
        fire_write(lA, rowsA, semWA)
        fire_gather(jnp.minimum(lA + 2, L - 1), rowsA, semA)
        wait_gather(rowsB, semB)

        @pl.when(i > 0)
        def _():
            wait_write(rowsB, semWB)

        add_char(rowsB)
        fire_write(lB, rowsB, semWB)
        fire_gather(jnp.minimum(lB + 2, L - 1), rowsB, semB)
        return carry

    lax.fori_loop(0, L // 2, body, 0)

    wait_gather(rowsA, semA)
    wait_gather(rowsB, semB)
    wait_write(rowsA, semWA)
    wait_write(rowsB, semWB)


def _sc_gather_add(T, C, IDX, chars):
    mesh = plsc.VectorSubcoreMesh(core_axis_name="c", subcore_axis_name="s")
    f = functools.partial(
        pl.kernel,
        mesh=mesh,
        compiler_params=pltpu.CompilerParams(
            use_tc_tiling_on_sc=False,
            disable_bounds_checks=True,
        ),
        out_type=jax.ShapeDtypeStruct((LP, B, 2 * D), jnp.float32),
        scratch_types=[
            pltpu.VMEM((L, BT), jnp.int32),
            pltpu.VMEM((BT, D), jnp.float32),
            pltpu.VMEM((BT,), jnp.int32),
            pltpu.VMEM((BT, D), jnp.float32),
            pltpu.VMEM((BT, D), jnp.float32),
            pltpu.SemaphoreType.DMA,
            pltpu.SemaphoreType.DMA,
            pltpu.SemaphoreType.DMA,
            pltpu.SemaphoreType.DMA,
        ],
    )(_sc_body)
    return f(T, C, IDX, chars)


def _tr_body(y_ref, o_ref):
    for p in range(4):
        x = y_ref[p]
        o_ref[2 * p] = x[:, :D].T
        o_ref[2 * p + 1] = x[:, D:].T


def _tc_transpose(Y):
    return pl.pallas_call(
        _tr_body,
        grid=(LP // 4,),
        in_specs=[pl.BlockSpec((4, BC, 2 * D), lambda i: (i, 0, 0))],
        out_specs=pl.BlockSpec((8, D, BC), lambda i: (i, 0, 0)),
        out_shape=jax.ShapeDtypeStruct((L, D, B), jnp.float32),
    )(Y)


def kernel(text_seqs, chars, text_table, char_table, W, b):
    Wt = W[:D]
    Wc = W[D:]
    T = _fold_text_table(text_table, Wt)
    Cb = _fold_char_table(char_table, Wc, b.reshape(1, D))
    IDX = (
        text_seqs.astype(jnp.int32)
        .reshape(NW, BT, L)
        .transpose(0, 2, 1)
        .reshape(NW * L, BT)
    )
    Y = _sc_gather_add(T, Cb, IDX, chars.astype(jnp.int32))
    out_t = _tc_transpose(Y)
    return out_t.transpose(2, 0, 1)
